# baseline (speedup 1.0000x reference)
import jax
import jax.numpy as jnp
from jax import lax
from jax.experimental import pallas as pl
from jax.experimental.pallas import tpu as pltpu
from jax.experimental.pallas import tpu_sc as plsc

NUM_CORES = 2
NUM_SUBCORES = 16
LANES = 16
NW = NUM_CORES * NUM_SUBCORES
B = 16384
E = 64
BPW = B // NW
CHUNK = 128
NCHUNK = BPW // CHUNK


def _sc_body(uidx_hbm, midx_hbm, uemb_hbm, memb_hbm, ubias_hbm, mbias_hbm,
             partials_hbm, bsum_hbm,
             uidx_v, midx_v, urows_v, mrows_v, ub_v, mb_v, bsum_v, pacc_v,
             emb_sem, bias_sem):
    wid = lax.axis_index("s") * NUM_CORES + lax.axis_index("c")
    base = wid * BPW

    pltpu.sync_copy(uidx_hbm.at[pl.ds(wid * NCHUNK, NCHUNK)], uidx_v)
    pltpu.sync_copy(midx_hbm.at[pl.ds(wid * NCHUNK, NCHUNK)], midx_v)

    copies = []
    for j in range(NCHUNK):
        sl = pl.ds(j * CHUNK, CHUNK)
        copies.append(pltpu.async_copy(uemb_hbm.at[uidx_v.at[j]], urows_v.at[sl], emb_sem))
        copies.append(pltpu.async_copy(memb_hbm.at[midx_v.at[j]], mrows_v.at[sl], emb_sem))
        copies.append(pltpu.async_copy(ubias_hbm.at[uidx_v.at[j]], ub_v.at[sl], bias_sem))
        copies.append(pltpu.async_copy(mbias_hbm.at[midx_v.at[j]], mb_v.at[sl], bias_sem))
    for c in copies:
        c.wait()

    zero = jnp.zeros((LANES,), jnp.float32)

    def row_body(i, accs):
        out = []
        for j in range(E // LANES):
            sl = pl.ds(j * LANES, LANES)
            out.append(accs[j] + urows_v[i, sl] * mrows_v[i, sl])
        return tuple(out)

    accs = lax.fori_loop(0, BPW, row_body, (zero, zero, zero, zero))
    pacc_v[...] = (accs[0] + accs[1]) + (accs[2] + accs[3])
    pltpu.sync_copy(pacc_v, partials_hbm.at[wid])

    for k in range(BPW // LANES):
        sl = pl.ds(k * LANES, LANES)
        bsum_v[sl] = ub_v[sl] + mb_v[sl]
    pltpu.sync_copy(bsum_v, bsum_hbm.at[pl.ds(base, BPW)])


def _sc_call(uidx, midx, uemb, memb, ubias, mbias):
    mesh = plsc.VectorSubcoreMesh(core_axis_name="c", subcore_axis_name="s",
                                  num_cores=NUM_CORES, num_subcores=NUM_SUBCORES)
    return pl.kernel(
        _sc_body,
        out_type=(
            jax.ShapeDtypeStruct((NW, LANES), jnp.float32),
            jax.ShapeDtypeStruct((B,), jnp.float32),
        ),
        mesh=mesh,
        compiler_params=pltpu.CompilerParams(use_tc_tiling_on_sc=False),
        scratch_types=[
            pltpu.VMEM((NCHUNK, CHUNK), jnp.int32),
            pltpu.VMEM((NCHUNK, CHUNK), jnp.int32),
            pltpu.VMEM((BPW, E), jnp.float32),
            pltpu.VMEM((BPW, E), jnp.float32),
            pltpu.VMEM((BPW,), jnp.float32),
            pltpu.VMEM((BPW,), jnp.float32),
            pltpu.VMEM((BPW,), jnp.float32),
            pltpu.VMEM((LANES,), jnp.float32),
            pltpu.SemaphoreType.DMA,
            pltpu.SemaphoreType.DMA,
        ],
    )(uidx, midx, uemb, memb, ubias, mbias)


def _tc_body(partials_ref, bsum_ref, out_ref):
    s = jnp.sum(partials_ref[...])
    out_ref[...] = jax.nn.sigmoid(bsum_ref[...] + s)


def _tc_call(partials, bsum2d):
    return pl.pallas_call(
        _tc_body,
        out_shape=jax.ShapeDtypeStruct(bsum2d.shape, jnp.float32),
    )(partials, bsum2d)


def kernel(inputs, user_emb, user_bias_tab, movie_emb, movie_bias_tab):
    uidx = inputs[:, 0].reshape(NW * NCHUNK, CHUNK)
    midx = inputs[:, 1].reshape(NW * NCHUNK, CHUNK)
    ubias = user_bias_tab.reshape(-1)
    mbias = movie_bias_tab.reshape(-1)
    partials, bsum = _sc_call(uidx, midx, user_emb, movie_emb, ubias, mbias)
    y = _tc_call(partials, bsum.reshape(128, 128))
    return y.reshape(B, 1)


# speedup vs baseline: 1.0004x; 1.0004x over previous
import jax
import jax.numpy as jnp
from jax import lax
from jax.experimental import pallas as pl
from jax.experimental.pallas import tpu as pltpu
from jax.experimental.pallas import tpu_sc as plsc

NUM_CORES = 2
NUM_SUBCORES = 16
LANES = 16
NW = NUM_CORES * NUM_SUBCORES
B = 16384
E = 64
BPW = B // NW
CHUNK = 128
NCHUNK = BPW // CHUNK


def _sc_body(uidx_hbm, midx_hbm, uemb_hbm, memb_hbm, ubias_hbm, mbias_hbm,
             partials_hbm, bsum_hbm,
             uidx_v, midx_v, urows_v, mrows_v, ub_v, mb_v, bsum_v, pacc_v,
             emb_sem, bias_sem):
    wid = lax.axis_index("s") * NUM_CORES + lax.axis_index("c")
    base = wid * BPW

    pltpu.sync_copy(uidx_hbm.at[pl.ds(wid * NCHUNK, NCHUNK)], uidx_v)
    pltpu.sync_copy(midx_hbm.at[pl.ds(wid * NCHUNK, NCHUNK)], midx_v)

    copies = []
    for j in range(NCHUNK):
        sl = pl.ds(j * CHUNK, CHUNK)
        copies.append(pltpu.async_copy(uemb_hbm.at[uidx_v.at[j]], urows_v.at[sl], emb_sem))
        copies.append(pltpu.async_copy(memb_hbm.at[midx_v.at[j]], mrows_v.at[sl], emb_sem))
        copies.append(pltpu.async_copy(ubias_hbm.at[uidx_v.at[j]], ub_v.at[sl], bias_sem))
        copies.append(pltpu.async_copy(mbias_hbm.at[midx_v.at[j]], mb_v.at[sl], bias_sem))
    for c in copies:
        c.wait()

    zero = jnp.zeros((LANES,), jnp.float32)

    def row_body(i, accs):
        out = []
        for j in range(E // LANES):
            sl = pl.ds(j * LANES, LANES)
            out.append(accs[j] + urows_v[i, sl] * mrows_v[i, sl])
        return tuple(out)

    accs = lax.fori_loop(0, BPW, row_body, (zero, zero, zero, zero))
    pacc_v[...] = (accs[0] + accs[1]) + (accs[2] + accs[3])
    pltpu.sync_copy(pacc_v, partials_hbm.at[wid])

    for k in range(BPW // LANES):
        sl = pl.ds(k * LANES, LANES)
        bsum_v[sl] = ub_v[sl] + mb_v[sl]
    pltpu.sync_copy(bsum_v, bsum_hbm.at[pl.ds(base, BPW)])


def _sc_call(uidx, midx, uemb, memb, ubias, mbias):
    mesh = plsc.VectorSubcoreMesh(core_axis_name="c", subcore_axis_name="s",
                                  num_cores=NUM_CORES, num_subcores=NUM_SUBCORES)
    return pl.kernel(
        _sc_body,
        out_type=(
            jax.ShapeDtypeStruct((NW, LANES), jnp.float32),
            jax.ShapeDtypeStruct((B,), jnp.float32),
        ),
        mesh=mesh,
        compiler_params=pltpu.CompilerParams(use_tc_tiling_on_sc=False),
        scratch_types=[
            pltpu.VMEM((NCHUNK, CHUNK), jnp.int32),
            pltpu.VMEM((NCHUNK, CHUNK), jnp.int32),
            pltpu.VMEM((BPW, E), jnp.float32),
            pltpu.VMEM((BPW, E), jnp.float32),
            pltpu.VMEM((BPW,), jnp.float32),
            pltpu.VMEM((BPW,), jnp.float32),
            pltpu.VMEM((BPW,), jnp.float32),
            pltpu.VMEM((LANES,), jnp.float32),
            pltpu.SemaphoreType.DMA,
            pltpu.SemaphoreType.DMA,
        ],
    )(uidx, midx, uemb, memb, ubias, mbias)


def _tc_body(partials_ref, bsum_ref, out_ref):
    s = jnp.sum(partials_ref[...])
    out_ref[...] = jax.nn.sigmoid(bsum_ref[...] + s)


def _tc_call(partials, bsum2d):
    return pl.pallas_call(
        _tc_body,
        out_shape=jax.ShapeDtypeStruct(bsum2d.shape, jnp.float32),
    )(partials, bsum2d)


def kernel(inputs, user_emb, user_bias_tab, movie_emb, movie_bias_tab):
    uidx = inputs[:, 0].reshape(NW * NCHUNK, CHUNK)
    midx = inputs[:, 1].reshape(NW * NCHUNK, CHUNK)
    uflat, mflat = lax.optimization_barrier(
        (user_emb.reshape(-1), movie_emb.reshape(-1)))
    uemb = uflat.reshape(100000, E)
    memb = mflat.reshape(100000, E)
    ubias = user_bias_tab.reshape(-1)
    mbias = movie_bias_tab.reshape(-1)
    partials, bsum = _sc_call(uidx, midx, uemb, memb, ubias, mbias)
    y = _tc_call(partials, bsum.reshape(128, 128))
    return y.reshape(B, 1)
